# baseline (speedup 1.0000x reference)
import dataclasses

import jax
import jax.numpy as jnp
from jax import lax
from jax.experimental import pallas as pl
from jax.experimental.pallas import tpu as pltpu
from jax.experimental.pallas import tpu_sc as plsc

BS, NA, NH = 1024, 16, 26
VOCAB, D = 100000, 32
NUM_IDX = BS * NA * NH
LANES = 128
PACK = LANES // D

VB = 1024
WB = VB // PACK
NVB = -(-VOCAB // VB)
WPH = NVB * WB
VOCAB_PAD = WPH * PACK

NC, NS = 2, 16
NW = NC * NS
PER_W = NUM_IDX // NW
CHUNK = 128
N_CHUNK = PER_W // CHUNK


def _transpose_tables(tables):
    t2 = jnp.transpose(tables, (0, 2, 1))

    def body(x_ref, o_ref):
        x = x_ref[0]
        x3 = x.reshape(D, WB, PACK)
        o_ref[0] = jnp.transpose(x3, (1, 2, 0)).reshape(WB, LANES)

    return pl.pallas_call(
        body,
        grid=(NH, NVB),
        in_specs=[
            pl.BlockSpec((1, D, VB), lambda h, b: (h, 0, b)),
        ],
        out_specs=pl.BlockSpec((1, WB, LANES), lambda h, b: (h, b, 0)),
        out_shape=jax.ShapeDtypeStruct((NH, WPH, LANES), jnp.float32),
    )(t2)


def kernel(prev_act, tables):
    wide = _transpose_tables(tables)
    flat_tables = wide.reshape(NH * VOCAB_PAD, D)
    offs = jnp.arange(NH, dtype=jnp.int32) * VOCAB_PAD
    g = (prev_act.reshape(BS * NA, NH) + offs[None, :]).reshape(
        NW, N_CHUNK, CHUNK
    )

    mesh = plsc.VectorSubcoreMesh(core_axis_name="c", subcore_axis_name="s")
    cp = pltpu.CompilerParams(use_tc_tiling_on_sc=False)
    if "needs_layout_passes" in pltpu.CompilerParams.__dataclass_fields__:
        cp = dataclasses.replace(cp, needs_layout_passes=False)

    @pl.kernel(
        out_type=jax.ShapeDtypeStruct((NUM_IDX, D), jnp.float32),
        mesh=mesh,
        compiler_params=cp,
        scratch_types=[
            pltpu.VMEM((4, 1, CHUNK), jnp.int32),
            pltpu.VMEM((2, CHUNK, D), jnp.float32),
            pltpu.SemaphoreType.DMA,
            pltpu.SemaphoreType.DMA,
            pltpu.SemaphoreType.DMA,
            pltpu.SemaphoreType.DMA,
            pltpu.SemaphoreType.DMA,
            pltpu.SemaphoreType.DMA,
            pltpu.SemaphoreType.DMA,
            pltpu.SemaphoreType.DMA,
        ],
    )
    def gather_kernel(table_hbm, g_hbm, out_hbm, idx_v, out_v,
                      isem0, isem1, isem2, isem3, gsem0, gsem1, wsem0, wsem1):
        isems = (isem0, isem1, isem2, isem3)
        gsems = (gsem0, gsem1)
        wsems = (wsem0, wsem1)
        wid = lax.axis_index("s") * NC + lax.axis_index("c")
        base = wid * PER_W

        def istart(c, r):
            pltpu.async_copy(g_hbm.at[wid, c], idx_v.at[r, 0], isems[r])

        def iwait(r):
            pltpu.make_async_copy(
                g_hbm.at[wid, 0], idx_v.at[r, 0], isems[r]
            ).wait()

        def gstart(r, b):
            pltpu.async_copy(
                table_hbm.at[idx_v.at[r, 0]], out_v.at[b], gsems[b]
            )

        def gwait(b):
            pltpu.make_async_copy(
                table_hbm.at[idx_v.at[0, 0]], out_v.at[b], gsems[b]
            ).wait()

        def wstart(c, b):
            pltpu.async_copy(
                out_v.at[b], out_hbm.at[pl.ds(base + c * CHUNK, CHUNK)],
                wsems[b]
            )

        def wwait(b):
            pltpu.make_async_copy(
                out_v.at[b], out_hbm.at[pl.ds(base, CHUNK)], wsems[b]
            ).wait()

        for r in range(4):
            istart(r, r)
        iwait(0)
        gstart(0, 0)
        iwait(1)
        gstart(1, 1)
        gwait(0)
        wstart(0, 0)
        istart(4, 0)
        iwait(2)
        wwait(0)
        gstart(2, 0)
        gwait(1)
        wstart(1, 1)
        istart(5, 1)
        iwait(3)
        wwait(1)
        gstart(3, 1)
        gwait(0)
        wstart(2, 0)
        istart(6, 2)
        iwait(0)
        wwait(0)
        gstart(0, 0)
        gwait(1)
        wstart(3, 1)
        istart(7, 3)

        @pl.loop(1, N_CHUNK // 4 - 1)
        def _(q):
            c0 = 4 * q
            for r in range(4):
                c = c0 + r
                iwait((r + 1) % 4)
                wwait((r + 1) % 2)
                gstart((r + 1) % 4, (r + 1) % 2)
                gwait(r % 2)
                wstart(c, r % 2)
                istart(c + 4, r)

        ce = N_CHUNK - 4
        for r in range(3):
            c = ce + r
            iwait((r + 1) % 4)
            wwait((r + 1) % 2)
            gstart((r + 1) % 4, (r + 1) % 2)
            gwait(r % 2)
            wstart(c, r % 2)
        gwait(1)
        wstart(N_CHUNK - 1, 1)
        wwait(0)
        wwait(1)

    out = gather_kernel(flat_tables, g)
    return out.reshape(BS, NA, NH * D)


# speedup vs baseline: 4.0828x; 4.0828x over previous
import dataclasses

import jax
import jax.numpy as jnp
from jax import lax
from jax.experimental import pallas as pl
from jax.experimental.pallas import tpu as pltpu
from jax.experimental.pallas import tpu_sc as plsc

BS, NA, NH = 1024, 16, 26
VOCAB, D = 100000, 32
NUM_IDX = BS * NA * NH
LANES = 128
PACK = LANES // D

WB = 256
NVB = 98
WPH = NVB * WB
VOCAB_PAD = WPH * PACK

NC, NS = 2, 16
NW = NC * NS
PER_W = NUM_IDX // NW
CHUNK = 128
N_CHUNK = PER_W // CHUNK


def _transpose_tables(tables):
    t2 = jnp.transpose(tables, (0, 2, 1))

    def body(x0_ref, x1_ref, x2_ref, x3_ref, o_ref):
        z = jnp.concatenate(
            [x0_ref[0], x1_ref[0], x2_ref[0], x3_ref[0]], axis=0
        )
        o_ref[0] = jnp.transpose(z)

    def in_spec(k):
        return pl.BlockSpec((1, D, WB), lambda h, b, k=k: (h, 0, b + k * NVB))

    return pl.pallas_call(
        body,
        grid=(NH, NVB),
        in_specs=[in_spec(k) for k in range(PACK)],
        out_specs=pl.BlockSpec((1, WB, LANES), lambda h, b: (h, b, 0)),
        out_shape=jax.ShapeDtypeStruct((NH, WPH, LANES), jnp.float32),
        compiler_params=pltpu.CompilerParams(
            dimension_semantics=("parallel", "parallel")
        ),
    )(t2, t2, t2, t2)


def kernel(prev_act, tables):
    wide = _transpose_tables(tables)
    flat_tables = wide.reshape(NH * VOCAB_PAD, D)
    offs = jnp.arange(NH, dtype=jnp.int32) * VOCAB_PAD
    r = prev_act.reshape(BS * NA, NH)
    q = r // WPH
    g = (offs[None, :] + (r - q * WPH) * PACK + q).reshape(
        NW, N_CHUNK, CHUNK
    )

    mesh = plsc.VectorSubcoreMesh(core_axis_name="c", subcore_axis_name="s")
    cp = pltpu.CompilerParams(use_tc_tiling_on_sc=False)
    if "needs_layout_passes" in pltpu.CompilerParams.__dataclass_fields__:
        cp = dataclasses.replace(cp, needs_layout_passes=False)

    @pl.kernel(
        out_type=jax.ShapeDtypeStruct((NUM_IDX, D), jnp.float32),
        mesh=mesh,
        compiler_params=cp,
        scratch_types=[
            pltpu.VMEM((4, 1, CHUNK), jnp.int32),
            pltpu.VMEM((2, CHUNK, D), jnp.float32),
            pltpu.SemaphoreType.DMA,
            pltpu.SemaphoreType.DMA,
            pltpu.SemaphoreType.DMA,
            pltpu.SemaphoreType.DMA,
            pltpu.SemaphoreType.DMA,
            pltpu.SemaphoreType.DMA,
            pltpu.SemaphoreType.DMA,
            pltpu.SemaphoreType.DMA,
        ],
    )
    def gather_kernel(table_hbm, g_hbm, out_hbm, idx_v, out_v,
                      isem0, isem1, isem2, isem3, gsem0, gsem1, wsem0, wsem1):
        isems = (isem0, isem1, isem2, isem3)
        gsems = (gsem0, gsem1)
        wsems = (wsem0, wsem1)
        wid = lax.axis_index("s") * NC + lax.axis_index("c")
        base = wid * PER_W

        def istart(c, r):
            pltpu.async_copy(g_hbm.at[wid, c], idx_v.at[r, 0], isems[r])

        def iwait(r):
            pltpu.make_async_copy(
                g_hbm.at[wid, 0], idx_v.at[r, 0], isems[r]
            ).wait()

        def gstart(r, b):
            pltpu.async_copy(
                table_hbm.at[idx_v.at[r, 0]], out_v.at[b], gsems[b]
            )

        def gwait(b):
            pltpu.make_async_copy(
                table_hbm.at[idx_v.at[0, 0]], out_v.at[b], gsems[b]
            ).wait()

        def wstart(c, b):
            pltpu.async_copy(
                out_v.at[b], out_hbm.at[pl.ds(base + c * CHUNK, CHUNK)],
                wsems[b]
            )

        def wwait(b):
            pltpu.make_async_copy(
                out_v.at[b], out_hbm.at[pl.ds(base, CHUNK)], wsems[b]
            ).wait()

        for r in range(4):
            istart(r, r)
        iwait(0)
        gstart(0, 0)
        iwait(1)
        gstart(1, 1)
        gwait(0)
        wstart(0, 0)
        istart(4, 0)
        iwait(2)
        wwait(0)
        gstart(2, 0)
        gwait(1)
        wstart(1, 1)
        istart(5, 1)
        iwait(3)
        wwait(1)
        gstart(3, 1)
        gwait(0)
        wstart(2, 0)
        istart(6, 2)
        iwait(0)
        wwait(0)
        gstart(0, 0)
        gwait(1)
        wstart(3, 1)
        istart(7, 3)

        @pl.loop(1, N_CHUNK // 4 - 1)
        def _(q):
            c0 = 4 * q
            for r in range(4):
                c = c0 + r
                iwait((r + 1) % 4)
                wwait((r + 1) % 2)
                gstart((r + 1) % 4, (r + 1) % 2)
                gwait(r % 2)
                wstart(c, r % 2)
                istart(c + 4, r)

        ce = N_CHUNK - 4
        for r in range(3):
            c = ce + r
            iwait((r + 1) % 4)
            wwait((r + 1) % 2)
            gstart((r + 1) % 4, (r + 1) % 2)
            gwait(r % 2)
            wstart(c, r % 2)
        gwait(1)
        wstart(N_CHUNK - 1, 1)
        wwait(0)
        wwait(1)

    out = gather_kernel(flat_tables, g)
    return out.reshape(BS, NA, NH * D)


# speedup vs baseline: 5.1288x; 1.2562x over previous
import dataclasses

import jax
import jax.numpy as jnp
from jax import lax
from jax.experimental import pallas as pl
from jax.experimental.pallas import tpu as pltpu
from jax.experimental.pallas import tpu_sc as plsc

BS, NA, NH = 1024, 16, 26
VOCAB, D = 100000, 32
NUM_IDX = BS * NA * NH

NC, NS = 2, 16
NW = NC * NS
PER_W = NUM_IDX // NW
CHUNK = 128
N_CHUNK = PER_W // CHUNK


def kernel(prev_act, tables):
    flat_tables = tables.reshape(NH * VOCAB, D)
    offs = jnp.arange(NH, dtype=jnp.int32) * VOCAB
    g = (prev_act.reshape(BS * NA, NH) + offs[None, :]).reshape(
        NW, N_CHUNK, CHUNK
    )

    mesh = plsc.VectorSubcoreMesh(core_axis_name="c", subcore_axis_name="s")
    cp = pltpu.CompilerParams(use_tc_tiling_on_sc=False)
    if "needs_layout_passes" in pltpu.CompilerParams.__dataclass_fields__:
        cp = dataclasses.replace(cp, needs_layout_passes=False)

    @pl.kernel(
        out_type=jax.ShapeDtypeStruct((NUM_IDX, D), jnp.float32),
        mesh=mesh,
        compiler_params=cp,
        scratch_types=[
            pltpu.VMEM((4, 1, CHUNK), jnp.int32),
            pltpu.VMEM((2, CHUNK, D), jnp.float32),
            pltpu.SemaphoreType.DMA,
            pltpu.SemaphoreType.DMA,
            pltpu.SemaphoreType.DMA,
            pltpu.SemaphoreType.DMA,
            pltpu.SemaphoreType.DMA,
            pltpu.SemaphoreType.DMA,
            pltpu.SemaphoreType.DMA,
            pltpu.SemaphoreType.DMA,
        ],
    )
    def gather_kernel(table_hbm, g_hbm, out_hbm, idx_v, out_v,
                      isem0, isem1, isem2, isem3, gsem0, gsem1, wsem0, wsem1):
        isems = (isem0, isem1, isem2, isem3)
        gsems = (gsem0, gsem1)
        wsems = (wsem0, wsem1)
        wid = lax.axis_index("s") * NC + lax.axis_index("c")
        base = wid * PER_W

        def istart(c, r):
            pltpu.async_copy(g_hbm.at[wid, c], idx_v.at[r, 0], isems[r])

        def iwait(r):
            pltpu.make_async_copy(
                g_hbm.at[wid, 0], idx_v.at[r, 0], isems[r]
            ).wait()

        def gstart(r, b):
            pltpu.async_copy(
                table_hbm.at[idx_v.at[r, 0]], out_v.at[b], gsems[b]
            )

        def gwait(b):
            pltpu.make_async_copy(
                table_hbm.at[idx_v.at[0, 0]], out_v.at[b], gsems[b]
            ).wait()

        def wstart(c, b):
            pltpu.async_copy(
                out_v.at[b], out_hbm.at[pl.ds(base + c * CHUNK, CHUNK)],
                wsems[b]
            )

        def wwait(b):
            pltpu.make_async_copy(
                out_v.at[b], out_hbm.at[pl.ds(base, CHUNK)], wsems[b]
            ).wait()

        for r in range(4):
            istart(r, r)
        iwait(0)
        gstart(0, 0)
        iwait(1)
        gstart(1, 1)
        gwait(0)
        wstart(0, 0)
        istart(4, 0)
        iwait(2)
        wwait(0)
        gstart(2, 0)
        gwait(1)
        wstart(1, 1)
        istart(5, 1)
        iwait(3)
        wwait(1)
        gstart(3, 1)
        gwait(0)
        wstart(2, 0)
        istart(6, 2)
        iwait(0)
        wwait(0)
        gstart(0, 0)
        gwait(1)
        wstart(3, 1)
        istart(7, 3)

        @pl.loop(1, N_CHUNK // 4 - 1)
        def _(q):
            c0 = 4 * q
            for r in range(4):
                c = c0 + r
                iwait((r + 1) % 4)
                wwait((r + 1) % 2)
                gstart((r + 1) % 4, (r + 1) % 2)
                gwait(r % 2)
                wstart(c, r % 2)
                istart(c + 4, r)

        ce = N_CHUNK - 4
        for r in range(3):
            c = ce + r
            iwait((r + 1) % 4)
            wwait((r + 1) % 2)
            gstart((r + 1) % 4, (r + 1) % 2)
            gwait(r % 2)
            wstart(c, r % 2)
        gwait(1)
        wstart(N_CHUNK - 1, 1)
        wwait(0)
        wwait(1)

    out = gather_kernel(flat_tables, g)
    return out.reshape(BS, NA, NH * D)
